# Initial kernel scaffold; baseline (speedup 1.0000x reference)
import functools

import jax
import jax.numpy as jnp
from jax import lax
from jax.experimental import pallas as pl
from jax.experimental.pallas import tpu as pltpu
from jax.experimental.pallas import tpu_sc as plsc

_N = 10000
_E = 320000
_D = 128
_NC = 2
_NS = 16
_NW = _NC * _NS
_NPAD = 10240
_CW = 16
_EPW = _E // _NW
_K = 80
_NCHUNK = _EPW // _K
_RPT = _NPAD // _NS


def _sc_segment_sum(x, row, col):
    mesh = plsc.VectorSubcoreMesh(core_axis_name="c", subcore_axis_name="s")

    @functools.partial(
        pl.kernel,
        out_type=[
            jax.ShapeDtypeStruct((_NC, _NPAD, _D), jnp.float32),
            jax.ShapeDtypeStruct((_NC, _NPAD, _CW), jnp.float32),
        ],
        mesh=mesh,
        scratch_types=[
            pltpu.VMEM((_K,), jnp.int32),
            pltpu.VMEM((_K,), jnp.int32),
            pltpu.VMEM((_K, _D), jnp.float32),
            pltpu.VMEM((_K, _CW), jnp.float32),
            pltpu.VMEM((16, _D), jnp.float32),
            pltpu.VMEM((16, _CW), jnp.float32),
            pltpu.VMEM_SHARED((_NPAD, _D), jnp.float32),
            pltpu.VMEM_SHARED((_NPAD, _CW), jnp.float32),
        ],
    )
    def sc_kernel(x_hbm, row_hbm, col_hbm, sum_out, cnt_out,
                  ridx, cidx, gbuf, ones, zrow, zcnt, acc, cnt):
        c = lax.axis_index("c")
        s = lax.axis_index("s")
        wid = c * _NS + s

        zero16 = jnp.zeros((16,), jnp.float32)
        one16 = jnp.ones((16,), jnp.float32)
        for r in range(16):
            for q in range(_D // 16):
                zrow.at[r, pl.ds(q * 16, 16)][...] = zero16
            zcnt.at[r, pl.ds(0, 16)][...] = zero16
        for r in range(_K):
            ones.at[r, pl.ds(0, 16)][...] = one16

        lo = s * _RPT

        @pl.loop(0, _RPT, step=16)
        def _(j):
            pltpu.sync_copy(zrow, acc.at[pl.ds(lo + j, 16)])
            pltpu.sync_copy(zcnt, cnt.at[pl.ds(lo + j, 16)])

        plsc.subcore_barrier()

        ebase = wid * _EPW

        @pl.loop(0, _NCHUNK)
        def _(j):
            off = ebase + j * _K
            pltpu.sync_copy(row_hbm.at[pl.ds(off, _K)], ridx)
            pltpu.sync_copy(col_hbm.at[pl.ds(off, _K)], cidx)
            pltpu.sync_copy(x_hbm.at[ridx], gbuf)
            pltpu.sync_copy(gbuf, acc.at[cidx], add=True)
            pltpu.sync_copy(ones, cnt.at[cidx], add=True)

        plsc.subcore_barrier()

        pltpu.sync_copy(acc.at[pl.ds(lo, _RPT)], sum_out.at[c, pl.ds(lo, _RPT)])
        pltpu.sync_copy(cnt.at[pl.ds(lo, _RPT)], cnt_out.at[c, pl.ds(lo, _RPT)])

    return sc_kernel(x, row, col)


def _tc_finish(x_pad, w_t, b2, sums, cnts):
    blk = 1024

    def body(x_ref, wt_ref, b_ref, s_ref, c_ref, o_ref):
        ssum = s_ref[0] + s_ref[1]
        cc = c_ref[0, :, 0:1] + c_ref[1, :, 0:1]
        m = jnp.where(cc > 0.0, ssum / jnp.maximum(cc, 1.0), x_ref[...])
        o_ref[...] = jnp.dot(
            m, wt_ref[...], preferred_element_type=jnp.float32,
            precision=lax.Precision.HIGHEST,
        ) + b_ref[...]

    return pl.pallas_call(
        body,
        grid=(_NPAD // blk,),
        in_specs=[
            pl.BlockSpec((blk, _D), lambda i: (i, 0)),
            pl.BlockSpec((_D, _D), lambda i: (0, 0)),
            pl.BlockSpec((1, _D), lambda i: (0, 0)),
            pl.BlockSpec((_NC, blk, _D), lambda i: (0, i, 0)),
            pl.BlockSpec((_NC, blk, _CW), lambda i: (0, i, 0)),
        ],
        out_specs=pl.BlockSpec((blk, _D), lambda i: (i, 0)),
        out_shape=jax.ShapeDtypeStruct((_NPAD, _D), jnp.float32),
    )(x_pad, w_t, b2, sums, cnts)


@jax.jit
def kernel(x, edge_index, W, b):
    row = edge_index[0]
    col = edge_index[1]
    sums, cnts = _sc_segment_sum(x, row, col)
    x_pad = jnp.pad(x, ((0, _NPAD - _N), (0, 0)))
    out_pad = _tc_finish(x_pad, W.T, b.reshape(1, _D), sums, cnts)
    return out_pad[:_N]


# speedup vs baseline: 6.1840x; 6.1840x over previous
import dataclasses
import functools

import jax
import jax.numpy as jnp
from jax import lax
from jax.experimental import pallas as pl
from jax.experimental.pallas import tpu as pltpu
from jax.experimental.pallas import tpu_sc as plsc

_N = 10000
_E = 320000
_D = 128
_NC = 2
_NS = 16
_NW = _NC * _NS
_NPAD = 10240
_EPW = _E // _NW
_K = 80
_NCHUNK = _EPW // _K
_RPT = _NPAD // _NS


def _sc_compiler_params():
    cp = pltpu.CompilerParams()
    if "needs_layout_passes" in pltpu.CompilerParams.__dataclass_fields__:
        cp = dataclasses.replace(cp, needs_layout_passes=False)
    return cp


def _sc_segment_sum(x, row, col):
    mesh = plsc.VectorSubcoreMesh(core_axis_name="c", subcore_axis_name="s")

    @functools.partial(
        pl.kernel,
        compiler_params=_sc_compiler_params(),
        out_type=[
            jax.ShapeDtypeStruct((_NPAD, _D), jnp.float32),
            jax.ShapeDtypeStruct((_NPAD, _D), jnp.float32),
            jax.ShapeDtypeStruct((_NW * _NPAD,), jnp.float32),
        ],
        mesh=mesh,
        scratch_types=[
            pltpu.VMEM((_K,), jnp.int32),
            pltpu.VMEM((_K,), jnp.int32),
            pltpu.VMEM((_K, _D), jnp.float32),
            pltpu.VMEM((16, _D), jnp.float32),
            pltpu.VMEM((_NPAD,), jnp.float32),
            pltpu.VMEM_SHARED((_NPAD, _D), jnp.float32),
        ],
    )
    def sc_kernel(x_hbm, row_hbm, col_hbm, sum0_out, sum1_out, cnt_out,
                  ridx, cidx, gbuf, zrow, hist, acc):
        c = lax.axis_index("c")
        s = lax.axis_index("s")
        wid = c * _NS + s

        zero16 = jnp.zeros((16,), jnp.float32)
        one16 = jnp.ones((16,), jnp.float32)
        for r in range(16):
            for q in range(_D // 16):
                zrow.at[r, pl.ds(q * 16, 16)][...] = zero16

        @pl.loop(0, _NPAD, step=16)
        def _(j):
            hist[pl.ds(j, 16)] = zero16

        lo = s * _RPT

        @pl.loop(0, _RPT, step=16)
        def _(j):
            pltpu.sync_copy(zrow, acc.at[pl.ds(lo + j, 16)])

        plsc.subcore_barrier()

        ebase = wid * _EPW

        @pl.loop(0, _NCHUNK)
        def _(j):
            off = ebase + j * _K
            pltpu.sync_copy(row_hbm.at[pl.ds(off, _K)], ridx)
            pltpu.sync_copy(col_hbm.at[pl.ds(off, _K)], cidx)
            pltpu.sync_copy(x_hbm.at[ridx], gbuf)
            pltpu.sync_copy(gbuf, acc.at[cidx], add=True)
            for q in range(_K // 16):
                idxv = cidx[pl.ds(q * 16, 16)]
                plsc.addupdate_scatter(hist, [idxv], one16)

        plsc.subcore_barrier()

        @pl.loop(0, _RPT, step=_K)
        def _(j):
            pltpu.sync_copy(acc.at[pl.ds(lo + j, _K)], gbuf)

            @pl.when(c == 0)
            def _():
                pltpu.sync_copy(gbuf, sum0_out.at[pl.ds(lo + j, _K)])

            @pl.when(c == 1)
            def _():
                pltpu.sync_copy(gbuf, sum1_out.at[pl.ds(lo + j, _K)])

        pltpu.sync_copy(hist, cnt_out.at[pl.ds(wid * _NPAD, _NPAD)])

    return sc_kernel(x, row, col)


def _tc_finish(x_pad, w_t, b2, sum0, sum1, cnt_t):
    blk = 1024

    def body(x_ref, wt_ref, b_ref, s0_ref, s1_ref, c_ref, o_ref):
        ssum = s0_ref[...] + s1_ref[...]
        cc = jnp.sum(c_ref[...], axis=1, keepdims=True)
        m = jnp.where(cc > 0.0, ssum / jnp.maximum(cc, 1.0), x_ref[...])
        o_ref[...] = jnp.dot(
            m, wt_ref[...], preferred_element_type=jnp.float32,
            precision=lax.Precision.HIGHEST,
        ) + b_ref[...]

    return pl.pallas_call(
        body,
        grid=(_NPAD // blk,),
        in_specs=[
            pl.BlockSpec((blk, _D), lambda i: (i, 0)),
            pl.BlockSpec((_D, _D), lambda i: (0, 0)),
            pl.BlockSpec((1, _D), lambda i: (0, 0)),
            pl.BlockSpec((blk, _D), lambda i: (i, 0)),
            pl.BlockSpec((blk, _D), lambda i: (i, 0)),
            pl.BlockSpec((blk, _NW), lambda i: (i, 0)),
        ],
        out_specs=pl.BlockSpec((blk, _D), lambda i: (i, 0)),
        out_shape=jax.ShapeDtypeStruct((_NPAD, _D), jnp.float32),
    )(x_pad, w_t, b2, sum0, sum1, cnt_t)


@jax.jit
def kernel(x, edge_index, W, b):
    row = edge_index[0]
    col = edge_index[1]
    sum0, sum1, cnth = _sc_segment_sum(x, row, col)
    cnt_t = cnth.reshape(_NW, _NPAD).T
    x_pad = jnp.pad(x, ((0, _NPAD - _N), (0, 0)))
    out_pad = _tc_finish(x_pad, W.T, b.reshape(1, _D), sum0, sum1, cnt_t)
    return out_pad[:_N]


# speedup vs baseline: 9.6300x; 1.5572x over previous
import dataclasses
import functools

import jax
import jax.numpy as jnp
from jax import lax
from jax.experimental import pallas as pl
from jax.experimental.pallas import tpu as pltpu
from jax.experimental.pallas import tpu_sc as plsc

_N = 10000
_E = 320000
_D = 128
_NC = 2
_NS = 16
_NW = _NC * _NS
_NPAD = 10240
_EPW = _E // _NW
_K = 80
_NCHUNK = _EPW // _K
_RPT = _NPAD // _NS


def _sc_compiler_params():
    cp = pltpu.CompilerParams()
    if "needs_layout_passes" in pltpu.CompilerParams.__dataclass_fields__:
        cp = dataclasses.replace(cp, needs_layout_passes=False)
    return cp


def _sc_segment_sum(x, row, col):
    mesh = plsc.VectorSubcoreMesh(core_axis_name="c", subcore_axis_name="s")

    @functools.partial(
        pl.kernel,
        compiler_params=_sc_compiler_params(),
        out_type=[
            jax.ShapeDtypeStruct((_NPAD, _D), jnp.float32),
            jax.ShapeDtypeStruct((_NPAD, _D), jnp.float32),
            jax.ShapeDtypeStruct((_NW * _NPAD,), jnp.float32),
        ],
        mesh=mesh,
        scratch_types=[
            pltpu.VMEM((_K,), jnp.int32),
            pltpu.VMEM((_K,), jnp.int32),
            pltpu.VMEM((_K,), jnp.int32),
            pltpu.VMEM((_K,), jnp.int32),
            pltpu.VMEM((_K, _D), jnp.float32),
            pltpu.VMEM((_K, _D), jnp.float32),
            pltpu.VMEM((_NPAD,), jnp.float32),
            pltpu.VMEM_SHARED((_NPAD, _D), jnp.float32),
            pltpu.SemaphoreType.DMA,
            pltpu.SemaphoreType.DMA,
            pltpu.SemaphoreType.DMA,
        ],
    )
    def sc_kernel(x_hbm, row_hbm, col_hbm, sum0_out, sum1_out, cnt_out,
                  ridx0, cidx0, ridx1, cidx1, gbuf0, gbuf1, hist, acc,
                  sg0, sg1, sz):
        c = lax.axis_index("c")
        s = lax.axis_index("s")
        wid = c * _NS + s
        lo = s * _RPT
        ebase = wid * _EPW

        zero16 = jnp.zeros((16,), jnp.float32)
        one16 = jnp.ones((16,), jnp.float32)

        @pl.loop(0, _K)
        def _(r):
            for q in range(_D // 16):
                gbuf0.at[r, pl.ds(q * 16, 16)][...] = zero16

        for t in range(_RPT // _K):
            pltpu.async_copy(gbuf0, acc.at[pl.ds(lo + t * _K, _K)], sz)

        @pl.loop(0, _NPAD, step=16)
        def _(j):
            hist[pl.ds(j, 16)] = zero16

        for t in range(_RPT // _K):
            pltpu.make_async_copy(gbuf0, acc.at[pl.ds(lo + t * _K, _K)],
                                  sz).wait()

        plsc.subcore_barrier()

        def load_idx(j, ridx, cidx):
            off = ebase + j * _K
            pltpu.sync_copy(row_hbm.at[pl.ds(off, _K)], ridx)
            pltpu.sync_copy(col_hbm.at[pl.ds(off, _K)], cidx)

        def consume(ridx, cidx, gbuf, sg):
            pltpu.make_async_copy(x_hbm.at[ridx], gbuf, sg).wait()
            pltpu.sync_copy(gbuf, acc.at[cidx], add=True)
            for q in range(_K // 16):
                idxv = cidx[pl.ds(q * 16, 16)]
                plsc.addupdate_scatter(hist, [idxv], one16)

        load_idx(0, ridx0, cidx0)
        pltpu.async_copy(x_hbm.at[ridx0], gbuf0, sg0)
        load_idx(1, ridx1, cidx1)
        pltpu.async_copy(x_hbm.at[ridx1], gbuf1, sg1)

        @pl.loop(0, (_NCHUNK - 3) // 2)
        def _(h):
            consume(ridx0, cidx0, gbuf0, sg0)
            load_idx(2 * h + 2, ridx0, cidx0)
            pltpu.async_copy(x_hbm.at[ridx0], gbuf0, sg0)
            consume(ridx1, cidx1, gbuf1, sg1)
            load_idx(2 * h + 3, ridx1, cidx1)
            pltpu.async_copy(x_hbm.at[ridx1], gbuf1, sg1)

        consume(ridx0, cidx0, gbuf0, sg0)
        load_idx(_NCHUNK - 1, ridx0, cidx0)
        pltpu.async_copy(x_hbm.at[ridx0], gbuf0, sg0)
        consume(ridx1, cidx1, gbuf1, sg1)
        consume(ridx0, cidx0, gbuf0, sg0)

        plsc.subcore_barrier()

        pltpu.async_copy(hist, cnt_out.at[pl.ds(wid * _NPAD, _NPAD)], sz)

        def out_slice(j, gbuf):
            pltpu.sync_copy(acc.at[pl.ds(j, _K)], gbuf)

            @pl.when(c == 0)
            def _():
                pltpu.sync_copy(gbuf, sum0_out.at[pl.ds(j, _K)])

            @pl.when(c == 1)
            def _():
                pltpu.sync_copy(gbuf, sum1_out.at[pl.ds(j, _K)])

        @pl.loop(0, _RPT, step=_K)
        def _(j):
            out_slice(lo + j, gbuf0)

        pltpu.make_async_copy(hist, cnt_out.at[pl.ds(wid * _NPAD, _NPAD)],
                              sz).wait()

    return sc_kernel(x, row, col)


def _tc_finish(x_pad, w_t, b2, sum0, sum1, cnt_t):
    blk = 1024

    def body(x_ref, wt_ref, b_ref, s0_ref, s1_ref, c_ref, o_ref):
        ssum = s0_ref[...] + s1_ref[...]
        cc = jnp.sum(c_ref[...], axis=1, keepdims=True)
        m = jnp.where(cc > 0.0, ssum / jnp.maximum(cc, 1.0), x_ref[...])
        o_ref[...] = jnp.dot(
            m, wt_ref[...], preferred_element_type=jnp.float32,
            precision=lax.Precision.HIGHEST,
        ) + b_ref[...]

    return pl.pallas_call(
        body,
        grid=(_NPAD // blk,),
        in_specs=[
            pl.BlockSpec((blk, _D), lambda i: (i, 0)),
            pl.BlockSpec((_D, _D), lambda i: (0, 0)),
            pl.BlockSpec((1, _D), lambda i: (0, 0)),
            pl.BlockSpec((blk, _D), lambda i: (i, 0)),
            pl.BlockSpec((blk, _D), lambda i: (i, 0)),
            pl.BlockSpec((blk, _NW), lambda i: (i, 0)),
        ],
        out_specs=pl.BlockSpec((blk, _D), lambda i: (i, 0)),
        out_shape=jax.ShapeDtypeStruct((_NPAD, _D), jnp.float32),
    )(x_pad, w_t, b2, sum0, sum1, cnt_t)


@jax.jit
def kernel(x, edge_index, W, b):
    row = edge_index[0]
    col = edge_index[1]
    sum0, sum1, cnth = _sc_segment_sum(x, row, col)
    cnt_t = cnth.reshape(_NW, _NPAD).T
    x_pad = jnp.pad(x, ((0, _NPAD - _N), (0, 0)))
    out_pad = _tc_finish(x_pad, W.T, b.reshape(1, _D), sum0, sum1, cnt_t)
    return out_pad[:_N]
